# baseline (speedup 1.0000x reference)
import functools

import jax
import jax.numpy as jnp
from jax import lax
from jax.experimental import pallas as pl
from jax.experimental.pallas import tpu as pltpu
from jax.experimental.pallas import tpu_sc as plsc

B = 4096
S = 200
D = 128
H = 512
VOUT = 1000

_NC = 2
_NS = 16
NW = _NC * _NS
BPW = B // NW

S0 = 104
S1 = S - S0


def _pool_sc(texts, embed):
    mesh = plsc.VectorSubcoreMesh(core_axis_name="c", subcore_axis_name="s")

    @functools.partial(
        pl.kernel,
        out_type=jax.ShapeDtypeStruct((B * D,), jnp.float32),
        mesh=mesh,
        scratch_types=[
            pltpu.VMEM((BPW * S,), jnp.int32),
            pltpu.VMEM((S, D), jnp.float32),
            pltpu.VMEM((S, D), jnp.float32),
            pltpu.VMEM((S, D), jnp.float32),
            pltpu.VMEM((S, D), jnp.float32),
            pltpu.VMEM((256,), jnp.float32),
            pltpu.SemaphoreType.DMA,
            pltpu.SemaphoreType.DMA,
            pltpu.SemaphoreType.DMA,
            pltpu.SemaphoreType.DMA,
            pltpu.SemaphoreType.DMA,
        ],
    )
    def k(texts_hbm, embed_hbm, out_hbm, idx_v, r0, r1, r2, r3, stage_v,
          s0, s1, s2, s3, sem_o):
        wid = lax.axis_index("s") * _NC + lax.axis_index("c")
        base = wid * BPW

        pltpu.sync_copy(texts_hbm.at[pl.ds(base * S, BPW * S)], idx_v)

        bufs = ((r0, s0), (r1, s1), (r2, s2), (r3, s3))

        def fire(b, rows_ref, sem):
            off = pl.multiple_of(b * S, 8)
            pltpu.async_copy(
                embed_hbm.at[idx_v.at[pl.ds(off, S0)]],
                rows_ref.at[pl.ds(0, S0)], sem)
            pltpu.async_copy(
                embed_hbm.at[idx_v.at[pl.ds(off + S0, S1)]],
                rows_ref.at[pl.ds(S0, S1)], sem)

        def wait(rows_ref, sem):
            pltpu.make_async_copy(
                embed_hbm.at[pl.ds(0, S)], rows_ref, sem).wait()

        scale = jnp.float32(1.0 / S)

        def drain_out():
            pltpu.make_async_copy(
                out_hbm.at[pl.ds(0, D)], stage_v.at[pl.ds(0, D)],
                sem_o).wait()

        def accum_out(b, rows_ref):
            def body(s, accs):
                return tuple(accs[j] + rows_ref[s, pl.ds(j * 16, 16)]
                             for j in range(8))
            accs = lax.fori_loop(
                0, S, body,
                tuple(jnp.zeros((16,), jnp.float32) for _ in range(8)))
            slot = (b % 2) * D

            @pl.when(b >= 2)
            def _():
                drain_out()

            for j in range(8):
                stage_v[pl.ds(slot + j * 16, 16)] = accs[j] * scale
            pltpu.async_copy(
                stage_v.at[pl.ds(slot, D)],
                out_hbm.at[pl.ds((base + b) * D, D)], sem_o)

        for e in range(3):
            fire(e, *bufs[e])

        def loop_body(i, carry):
            b = i * 4
            for u in range(4):
                rows_ref, sem = bufs[u]
                wait(rows_ref, sem)
                nxt = b + u + 3

                @pl.when(nxt < BPW)
                def _(nxt=nxt, u=u):
                    fire(nxt, *bufs[(u + 3) % 4])

                accum_out(b + u, rows_ref)

            return carry

        lax.fori_loop(0, BPW // 4, loop_body, 0)
        drain_out()
        drain_out()

    return k(texts.reshape(B * S), embed)


def _mlp_body(p_ref, w1_ref, b1_ref, w2_ref, b2_ref, o_ref):
    h = jnp.dot(p_ref[...], w1_ref[...],
                preferred_element_type=jnp.float32) + b1_ref[...]
    h = jnp.maximum(h, 0.0)
    o_ref[...] = jnp.dot(h, w2_ref[...],
                         preferred_element_type=jnp.float32) + b2_ref[...]


def _mlp_tc(pooled, W1, b1, W2, b2):
    BM = 512
    return pl.pallas_call(
        _mlp_body,
        grid=(B // BM,),
        in_specs=[
            pl.BlockSpec((BM, D), lambda i: (i, 0)),
            pl.BlockSpec((D, H), lambda i: (0, 0)),
            pl.BlockSpec((1, H), lambda i: (0, 0)),
            pl.BlockSpec((H, VOUT), lambda i: (0, 0)),
            pl.BlockSpec((1, VOUT), lambda i: (0, 0)),
        ],
        out_specs=pl.BlockSpec((BM, VOUT), lambda i: (i, 0)),
        out_shape=jax.ShapeDtypeStruct((B, VOUT), jnp.float32),
    )(pooled, W1, b1.reshape(1, H), W2, b2.reshape(1, VOUT))


def kernel(texts, embed, W1, b1, W2, b2):
    pooled = _pool_sc(texts, embed).reshape(B, D)
    return _mlp_tc(pooled, W1, b1, W2, b2)


# speedup vs baseline: 1.0125x; 1.0125x over previous
import functools

import jax
import jax.numpy as jnp
from jax import lax
from jax.experimental import pallas as pl
from jax.experimental.pallas import tpu as pltpu
from jax.experimental.pallas import tpu_sc as plsc

B = 4096
S = 200
D = 128
H = 512
VOUT = 1000

_NC = 2
_NS = 16
NW = _NC * _NS
BPW = B // NW

S0 = 104
S1 = S - S0


def _pool_sc(texts, embed):
    mesh = plsc.VectorSubcoreMesh(core_axis_name="c", subcore_axis_name="s")

    @functools.partial(
        pl.kernel,
        out_type=jax.ShapeDtypeStruct((B * D,), jnp.float32),
        mesh=mesh,
        scratch_types=[
            pltpu.VMEM((BPW * S,), jnp.int32),
            pltpu.VMEM((S, D), jnp.float32),
            pltpu.VMEM((S, D), jnp.float32),
            pltpu.VMEM((S, D), jnp.float32),
            pltpu.VMEM((S, D), jnp.float32),
            pltpu.VMEM((256,), jnp.float32),
            pltpu.SemaphoreType.DMA,
            pltpu.SemaphoreType.DMA,
            pltpu.SemaphoreType.DMA,
            pltpu.SemaphoreType.DMA,
            pltpu.SemaphoreType.DMA,
        ],
    )
    def k(texts_hbm, embed_hbm, out_hbm, idx_v, r0, r1, r2, r3, stage_v,
          s0, s1, s2, s3, sem_o):
        wid = lax.axis_index("s") * _NC + lax.axis_index("c")
        base = wid * BPW

        pltpu.sync_copy(texts_hbm.at[pl.ds(base * S, BPW * S)], idx_v)

        bufs = ((r0, s0), (r1, s1), (r2, s2), (r3, s3))

        def fire(b, rows_ref, sem):
            off = pl.multiple_of(b * S, 8)
            pltpu.async_copy(
                embed_hbm.at[idx_v.at[pl.ds(off, S0)]],
                rows_ref.at[pl.ds(0, S0)], sem)
            pltpu.async_copy(
                embed_hbm.at[idx_v.at[pl.ds(off + S0, S1)]],
                rows_ref.at[pl.ds(S0, S1)], sem)

        def wait(rows_ref, sem):
            pltpu.make_async_copy(
                embed_hbm.at[pl.ds(0, S)], rows_ref, sem).wait()

        scale = jnp.float32(1.0 / S)

        def drain_out():
            pltpu.make_async_copy(
                out_hbm.at[pl.ds(0, D)], stage_v.at[pl.ds(0, D)],
                sem_o).wait()

        def accum_out(b, rows_ref):
            def body(s, accs):
                return tuple(accs[j] + rows_ref[s, pl.ds(j * 16, 16)]
                             for j in range(8))
            accs = lax.fori_loop(
                0, S, body,
                tuple(jnp.zeros((16,), jnp.float32) for _ in range(8)))
            slot = (b % 2) * D

            @pl.when(b >= 2)
            def _():
                drain_out()

            for j in range(8):
                stage_v[pl.ds(slot + j * 16, 16)] = accs[j] * scale
            pltpu.async_copy(
                stage_v.at[pl.ds(slot, D)],
                out_hbm.at[pl.ds((base + b) * D, D)], sem_o)

        for e in range(3):
            fire(e, *bufs[e])

        def loop_body(i, carry):
            b = i * 4
            for u in range(4):
                rows_ref, sem = bufs[u]
                wait(rows_ref, sem)
                accum_out(b + u, rows_ref)
                nxt = b + u + 3

                @pl.when(nxt < BPW)
                def _(nxt=nxt, u=u):
                    fire(nxt, *bufs[(u + 3) % 4])

            return carry

        lax.fori_loop(0, BPW // 4, loop_body, 0)
        drain_out()
        drain_out()

    return k(texts.reshape(B * S), embed)


def _mlp_body(p_ref, w1_ref, b1_ref, w2_ref, b2_ref, o_ref):
    h = jnp.dot(p_ref[...], w1_ref[...],
                preferred_element_type=jnp.float32) + b1_ref[...]
    h = jnp.maximum(h, 0.0)
    o_ref[...] = jnp.dot(h, w2_ref[...],
                         preferred_element_type=jnp.float32) + b2_ref[...]


def _mlp_tc(pooled, W1, b1, W2, b2):
    BM = 512
    return pl.pallas_call(
        _mlp_body,
        grid=(B // BM,),
        in_specs=[
            pl.BlockSpec((BM, D), lambda i: (i, 0)),
            pl.BlockSpec((D, H), lambda i: (0, 0)),
            pl.BlockSpec((1, H), lambda i: (0, 0)),
            pl.BlockSpec((H, VOUT), lambda i: (0, 0)),
            pl.BlockSpec((1, VOUT), lambda i: (0, 0)),
        ],
        out_specs=pl.BlockSpec((BM, VOUT), lambda i: (i, 0)),
        out_shape=jax.ShapeDtypeStruct((B, VOUT), jnp.float32),
    )(pooled, W1, b1.reshape(1, H), W2, b2.reshape(1, VOUT))


def kernel(texts, embed, W1, b1, W2, b2):
    pooled = _pool_sc(texts, embed).reshape(B, D)
    return _mlp_tc(pooled, W1, b1, W2, b2)
